# Initial kernel scaffold; baseline (speedup 1.0000x reference)
import jax
import jax.numpy as jnp
from jax.experimental import pallas as pl


def kernel(goods_ids, shop_ids, cate_ids, goods_prices, goods_table, shop_table, cate_table, price_table):
    raise NotImplementedError("write your pallas kernel here")


# speedup vs baseline: 3.0923x; 3.0923x over previous
import functools

import jax
import jax.numpy as jnp
from jax import lax
from jax.experimental import pallas as pl
from jax.experimental.pallas import tpu as pltpu
from jax.experimental.pallas import tpu_sc as plsc

B, L, D = 4096, 50, 64
N = B * L
NT = 4
DOUT = NT * D

_info = plsc.get_sparse_core_info()
NC, NS = _info.num_cores, _info.num_subcores
NW = NC * NS
N_PER_W = N // NW
CHUNK = 256
N_CHUNKS = N_PER_W // CHUNK


def _make_kernel():
    mesh = plsc.VectorSubcoreMesh(core_axis_name="c", subcore_axis_name="s")

    @functools.partial(
        pl.kernel,
        mesh=mesh,
        out_type=jax.ShapeDtypeStruct((N, DOUT), jnp.float32),
        scratch_types=[
            pltpu.VMEM((CHUNK,), jnp.int32),
            pltpu.VMEM((CHUNK,), jnp.int32),
            pltpu.VMEM((CHUNK,), jnp.int32),
            pltpu.VMEM((CHUNK,), jnp.int32),
            pltpu.VMEM((CHUNK, D), jnp.float32),
            pltpu.VMEM((CHUNK, D), jnp.float32),
            pltpu.VMEM((CHUNK, D), jnp.float32),
            pltpu.VMEM((CHUNK, D), jnp.float32),
            pltpu.SemaphoreType.DMA,
        ],
        compiler_params=pltpu.CompilerParams(use_tc_tiling_on_sc=False),
    )
    def k(gids, sids, cids, pids, gt, st, ct, pt, out,
          ig, is_, ic, ip, rg, rs, rc, rp, sem):
        wid = lax.axis_index("s") * NC + lax.axis_index("c")
        base0 = wid * N_PER_W

        def step(i, _):
            base = base0 + i * CHUNK
            pltpu.sync_copy(gids.at[pl.ds(base, CHUNK)], ig)
            pltpu.sync_copy(sids.at[pl.ds(base, CHUNK)], is_)
            pltpu.sync_copy(cids.at[pl.ds(base, CHUNK)], ic)
            pltpu.sync_copy(pids.at[pl.ds(base, CHUNK)], ip)
            c0 = pltpu.async_copy(gt.at[ig], rg, sem)
            c1 = pltpu.async_copy(st.at[is_], rs, sem)
            c2 = pltpu.async_copy(ct.at[ic], rc, sem)
            c3 = pltpu.async_copy(pt.at[ip], rp, sem)
            c0.wait()
            c1.wait()
            c2.wait()
            c3.wait()
            pltpu.sync_copy(rg, out.at[pl.ds(base, CHUNK), pl.ds(0 * D, D)])
            pltpu.sync_copy(rs, out.at[pl.ds(base, CHUNK), pl.ds(1 * D, D)])
            pltpu.sync_copy(rc, out.at[pl.ds(base, CHUNK), pl.ds(2 * D, D)])
            pltpu.sync_copy(rp, out.at[pl.ds(base, CHUNK), pl.ds(3 * D, D)])
            return 0

        lax.fori_loop(0, N_CHUNKS, step, 0)

    return k


_kern = _make_kernel()


def kernel(goods_ids, shop_ids, cate_ids, goods_prices,
           goods_table, shop_table, cate_table, price_table):
    out = _kern(goods_ids.reshape(N), shop_ids.reshape(N),
                cate_ids.reshape(N), goods_prices.reshape(N),
                goods_table, shop_table, cate_table, price_table)
    return out.reshape(B, L, DOUT)


# speedup vs baseline: 3.2026x; 1.0357x over previous
import functools

import jax
import jax.numpy as jnp
from jax import lax
from jax.experimental import pallas as pl
from jax.experimental.pallas import tpu as pltpu
from jax.experimental.pallas import tpu_sc as plsc

B, L, D = 4096, 50, 64
N = B * L
NT = 4
DOUT = NT * D

_info = plsc.get_sparse_core_info()
NC, NS = _info.num_cores, _info.num_subcores
NW = NC * NS
N_PER_W = N // NW
CHUNK = 200
N_CHUNKS = N_PER_W // CHUNK
PAIRS = N_CHUNKS // 2


def _make_kernel():
    mesh = plsc.VectorSubcoreMesh(core_axis_name="c", subcore_axis_name="s")

    rows_scratch = [pltpu.VMEM((CHUNK, D), jnp.float32)
                    for _ in range(2 * NT)]

    @functools.partial(
        pl.kernel,
        mesh=mesh,
        out_type=jax.ShapeDtypeStruct((N, DOUT), jnp.float32),
        scratch_types=[
            pltpu.VMEM((N_PER_W,), jnp.int32),
            pltpu.VMEM((N_PER_W,), jnp.int32),
            pltpu.VMEM((N_PER_W,), jnp.int32),
            pltpu.VMEM((N_PER_W,), jnp.int32),
            *rows_scratch,
            pltpu.SemaphoreType.DMA,
            pltpu.SemaphoreType.DMA,
            pltpu.SemaphoreType.DMA,
            pltpu.SemaphoreType.DMA,
            pltpu.SemaphoreType.DMA,
        ],
        compiler_params=pltpu.CompilerParams(use_tc_tiling_on_sc=False),
    )
    def k(gids, sids, cids, pids, gt, st, ct, pt, out,
          ig, is_, ic, ip,
          rg0, rs0, rc0, rp0, rg1, rs1, rc1, rp1,
          sem_i, sem_g0, sem_g1, sem_w0, sem_w1):
        wid = lax.axis_index("s") * NC + lax.axis_index("c")
        base0 = wid * N_PER_W

        idx_refs = (ig, is_, ic, ip)
        tables = (gt, st, ct, pt)
        rows = ((rg0, rs0, rc0, rp0), (rg1, rs1, rc1, rp1))
        sem_g = (sem_g0, sem_g1)
        sem_w = (sem_w0, sem_w1)

        ci = [pltpu.async_copy(src.at[pl.ds(base0, N_PER_W)], dst, sem_i)
              for src, dst in zip((gids, sids, cids, pids), idx_refs)]
        for c in ci:
            c.wait()

        def fire_gathers(b, chunk):
            off = chunk * CHUNK
            return [pltpu.async_copy(tables[t].at[idx_refs[t].at[pl.ds(off, CHUNK)]],
                                     rows[b][t], sem_g[b])
                    for t in range(NT)]

        def fire_writes(b, chunk):
            base = base0 + chunk * CHUNK
            for t in range(NT):
                pltpu.async_copy(
                    rows[b][t],
                    out.at[pl.ds(base, CHUNK), pl.ds(t * D, D)], sem_w[b])

        def drain_writes(b):
            for t in range(NT):
                pltpu.make_async_copy(
                    out.at[pl.ds(base0, CHUNK), pl.ds(t * D, D)],
                    rows[b][t], sem_w[b]).wait()

        def pair(j, _):
            @pl.when(j > 0)
            def _():
                drain_writes(0)
            g0 = fire_gathers(0, 2 * j)

            @pl.when(j > 0)
            def _():
                drain_writes(1)
            g1 = fire_gathers(1, 2 * j + 1)
            for c in g0:
                c.wait()
            fire_writes(0, 2 * j)
            for c in g1:
                c.wait()
            fire_writes(1, 2 * j + 1)
            return 0

        lax.fori_loop(0, PAIRS, pair, 0)
        drain_writes(0)
        drain_writes(1)

    return k


_kern = _make_kernel()


def kernel(goods_ids, shop_ids, cate_ids, goods_prices,
           goods_table, shop_table, cate_table, price_table):
    out = _kern(goods_ids.reshape(N), shop_ids.reshape(N),
                cate_ids.reshape(N), goods_prices.reshape(N),
                goods_table, shop_table, cate_table, price_table)
    return out.reshape(B, L, DOUT)
